# Initial kernel scaffold; baseline (speedup 1.0000x reference)
import jax
import jax.numpy as jnp
from jax.experimental import pallas as pl


def kernel(x, edge_index, W_gat, attn_l, attn_r, bias_gat, W_lin):
    raise NotImplementedError("write your pallas kernel here")


# speedup vs baseline: 21.0652x; 21.0652x over previous
import jax
import jax.numpy as jnp
from jax import lax
from jax.experimental import pallas as pl
from jax.experimental.pallas import tpu as pltpu
from jax.experimental.pallas import tpu_sc as plsc

N = 10000
E = 320000
D = 128
NC = 2
NS = 16
NW = NC * NS
EPW = E // NW
CH = 80
NCHUNK = EPW // CH
LN = 16


def _pre_body(x_ref, wg_ref, al_ref, ar_ref, z_ref, el_ref, er_ref, m_ref):
    z = lax.dot_general(x_ref[...], wg_ref[...], (((1,), (1,)), ((), ())),
                        preferred_element_type=jnp.float32,
                        precision=lax.Precision.HIGHEST)
    z_ref[...] = z
    el = jnp.sum(z * al_ref[...][None, :], axis=1)
    er = jnp.sum(z * ar_ref[...][None, :], axis=1)
    el_ref[...] = el
    er_ref[...] = er
    m = jnp.max(el) + jnp.max(er)
    m = jnp.where(m >= 0.0, m, 0.2 * m)
    m_ref[...] = jnp.full((LN,), m, jnp.float32)


def _post_body(ph_ref, pd_ref, b_ref, wl_ref, o_ref):
    hu = ph_ref[0] + ph_ref[1]
    den = pd_ref[0, :, 0:1] + pd_ref[1, :, 0:1]
    h = hu / (den + 1e-9) + b_ref[...][None, :]
    h = jnp.where(h >= 0.0, h, 0.01 * h)
    o_ref[...] = lax.dot_general(h, wl_ref[...], (((1,), (1,)), ((), ())),
                                 preferred_element_type=jnp.float32,
                                 precision=lax.Precision.HIGHEST)


def _edge_body(z_hbm, el_hbm, er_hbm, src_hbm, dst_hbm, m_hbm,
               outh_hbm, outd_hbm,
               el_v, er_v, m_v, src_v, dst_v, zrow_v, wrow_v, w_v,
               acch_sh, accd_sh, sem):
    cid = lax.axis_index("c")
    sid = lax.axis_index("s")
    wid = cid * NS + sid

    pltpu.sync_copy(el_hbm, el_v)
    pltpu.sync_copy(er_hbm, er_v)
    pltpu.sync_copy(m_hbm, m_v)
    mvec = m_v[...]
    lane = lax.iota(jnp.int32, LN)
    zero16 = jnp.zeros((LN,), jnp.float32)

    @pl.loop(0, CH)
    def _zero(j):
        for c in range(D // LN):
            zrow_v[j, pl.ds(c * LN, LN)] = zero16
        wrow_v[j, :] = zero16

    @pl.loop(sid, N // CH, step=NS)
    def _zcopy(g):
        pltpu.sync_copy(zrow_v, acch_sh.at[pl.ds(g * CH, CH)])
        pltpu.sync_copy(wrow_v, accd_sh.at[pl.ds(g * CH, CH)])

    plsc.subcore_barrier()

    ebase = wid * EPW

    @pl.loop(0, NCHUNK)
    def _chunk(ci):
        base = ebase + ci * CH
        pltpu.sync_copy(src_hbm.at[pl.ds(base, CH)], src_v)
        pltpu.sync_copy(dst_hbm.at[pl.ds(base, CH)], dst_v)
        gat = pltpu.async_copy(z_hbm.at[src_v], zrow_v, sem)

        @pl.loop(0, CH // LN)
        def _wgrp(g):
            si = src_v[pl.ds(g * LN, LN)]
            di = dst_v[pl.ds(g * LN, LN)]
            s = plsc.load_gather(el_v, [si]) + plsc.load_gather(er_v, [di])
            e = jnp.where(s >= 0.0, s, 0.2 * s)
            w_v[pl.ds(g * LN, LN)] = jnp.exp(e - mvec)

        gat.wait()

        @pl.loop(0, CH)
        def _scale(j):
            wv = jnp.full((LN,), w_v[pl.ds(j, LN)][0], jnp.float32)
            for c in range(D // LN):
                zrow_v[j, pl.ds(c * LN, LN)] = (
                    zrow_v[j, pl.ds(c * LN, LN)] * wv)
            wrow_v[j, :] = jnp.where(lane == 0, wv, zero16)

        pltpu.sync_copy(zrow_v, acch_sh.at[dst_v], add=True)
        pltpu.sync_copy(wrow_v, accd_sh.at[dst_v], add=True)

    plsc.subcore_barrier()

    @pl.loop(sid, N // CH, step=NS)
    def _out(g):
        pltpu.sync_copy(acch_sh.at[pl.ds(g * CH, CH)],
                        outh_hbm.at[cid, pl.ds(g * CH, CH)])
        pltpu.sync_copy(accd_sh.at[pl.ds(g * CH, CH)],
                        outd_hbm.at[cid, pl.ds(g * CH, CH)])


def kernel(x, edge_index, W_gat, attn_l, attn_r, bias_gat, W_lin):
    src = edge_index[0].astype(jnp.int32)
    dst = edge_index[1].astype(jnp.int32)

    z, el, er, m = pl.pallas_call(
        _pre_body,
        out_shape=[
            jax.ShapeDtypeStruct((N, D), jnp.float32),
            jax.ShapeDtypeStruct((N,), jnp.float32),
            jax.ShapeDtypeStruct((N,), jnp.float32),
            jax.ShapeDtypeStruct((LN,), jnp.float32),
        ],
    )(x, W_gat, attn_l, attn_r)

    mesh = plsc.VectorSubcoreMesh(core_axis_name="c", subcore_axis_name="s",
                                  num_cores=NC, num_subcores=NS)
    edge_kernel = pl.kernel(
        _edge_body,
        out_type=[
            jax.ShapeDtypeStruct((NC, N, D), jnp.float32),
            jax.ShapeDtypeStruct((NC, N, LN), jnp.float32),
        ],
        mesh=mesh,
        compiler_params=pltpu.CompilerParams(use_tc_tiling_on_sc=False,
                                             needs_layout_passes=False),
        scratch_types=[
            pltpu.VMEM((N,), jnp.float32),
            pltpu.VMEM((N,), jnp.float32),
            pltpu.VMEM((LN,), jnp.float32),
            pltpu.VMEM((CH,), jnp.int32),
            pltpu.VMEM((CH,), jnp.int32),
            pltpu.VMEM((CH, D), jnp.float32),
            pltpu.VMEM((CH, LN), jnp.float32),
            pltpu.VMEM((CH + LN,), jnp.float32),
            pltpu.VMEM_SHARED((N, D), jnp.float32),
            pltpu.VMEM_SHARED((N, LN), jnp.float32),
            pltpu.SemaphoreType.DMA,
        ],
    )
    parts_h, parts_d = edge_kernel(z, el, er, src, dst, m)

    out = pl.pallas_call(
        _post_body,
        out_shape=jax.ShapeDtypeStruct((N, D), jnp.float32),
    )(parts_h, parts_d, bias_gat, W_lin)
    return out


# speedup vs baseline: 23.0867x; 1.0960x over previous
import jax
import jax.numpy as jnp
from jax import lax
from jax.experimental import pallas as pl
from jax.experimental.pallas import tpu as pltpu
from jax.experimental.pallas import tpu_sc as plsc

N = 10000
E = 320000
D = 128
NC = 2
NS = 16
NW = NC * NS
EPW = E // NW
CH = 80
NCHUNK = EPW // CH
LN = 16


def _pre_body(x_ref, wg_ref, al_ref, ar_ref, z_ref, el_ref, er_ref, m_ref):
    z = lax.dot_general(x_ref[...], wg_ref[...], (((1,), (1,)), ((), ())),
                        preferred_element_type=jnp.float32,
                        precision=lax.Precision.HIGHEST)
    z_ref[...] = z
    el = jnp.sum(z * al_ref[...][None, :], axis=1)
    er = jnp.sum(z * ar_ref[...][None, :], axis=1)
    el_ref[...] = el
    er_ref[...] = er
    m = jnp.max(el) + jnp.max(er)
    m = jnp.where(m >= 0.0, m, 0.2 * m)
    m_ref[...] = jnp.full((LN,), m, jnp.float32)


def _post_body(ph_ref, pd_ref, b_ref, wl_ref, o_ref):
    hu = ph_ref[0] + ph_ref[1]
    den = pd_ref[0, :, 0:1] + pd_ref[1, :, 0:1]
    h = hu / (den + 1e-9) + b_ref[...][None, :]
    h = jnp.where(h >= 0.0, h, 0.01 * h)
    o_ref[...] = lax.dot_general(h, wl_ref[...], (((1,), (1,)), ((), ())),
                                 preferred_element_type=jnp.float32,
                                 precision=lax.Precision.HIGHEST)


def _edge_body(z_hbm, el_hbm, er_hbm, sd_hbm, m_hbm,
               outh_hbm, outd_hbm,
               el_v, er_v, m_v, sd_v, zrow_v, wrow_v, w_v,
               acch_sh, accd_sh, gsem, hsem, dsem):
    cid = lax.axis_index("c")
    sid = lax.axis_index("s")
    wid = cid * NS + sid

    pltpu.sync_copy(el_hbm, el_v)
    pltpu.sync_copy(er_hbm, er_v)
    pltpu.sync_copy(m_hbm, m_v)
    mvec = m_v[...]
    lane = lax.iota(jnp.int32, LN)
    zero16 = jnp.zeros((LN,), jnp.float32)

    @pl.loop(0, CH)
    def _zero(j):
        for c in range(D // LN):
            zrow_v[j, pl.ds(c * LN, LN)] = zero16
        wrow_v[j, :] = zero16

    @pl.loop(sid, N // CH, step=NS)
    def _zcopy(g):
        pltpu.sync_copy(zrow_v, acch_sh.at[pl.ds(g * CH, CH)])
        pltpu.sync_copy(wrow_v, accd_sh.at[pl.ds(g * CH, CH)])

    plsc.subcore_barrier()

    cbase = wid * NCHUNK

    @pl.loop(0, NCHUNK)
    def _chunk(ci):
        pltpu.sync_copy(sd_hbm.at[cbase + ci], sd_v)
        gat = pltpu.async_copy(z_hbm.at[sd_v.at[0]], zrow_v, gsem)

        @pl.loop(0, CH // LN)
        def _wgrp(g):
            si = sd_v[0, pl.ds(g * LN, LN)]
            di = sd_v[1, pl.ds(g * LN, LN)]
            s = plsc.load_gather(el_v, [si]) + plsc.load_gather(er_v, [di])
            e = jnp.where(s >= 0.0, s, 0.2 * s)
            w_v[pl.ds(g * LN, LN)] = jnp.exp(e - mvec)

        gat.wait()

        @pl.loop(0, CH)
        def _scale(j):
            wv = jnp.full((LN,), w_v[pl.ds(j, LN)][0], jnp.float32)
            for c in range(D // LN):
                zrow_v[j, pl.ds(c * LN, LN)] = (
                    zrow_v[j, pl.ds(c * LN, LN)] * wv)
            wrow_v[j, :] = jnp.where(lane == 0, wv, zero16)

        sc_h = pltpu.async_copy(zrow_v, acch_sh.at[sd_v.at[1]], hsem,
                                add=True)
        sc_d = pltpu.async_copy(wrow_v, accd_sh.at[sd_v.at[1]], dsem,
                                add=True)
        sc_h.wait()
        sc_d.wait()

    plsc.subcore_barrier()

    @pl.loop(sid, N // CH, step=NS)
    def _out(g):
        pltpu.sync_copy(acch_sh.at[pl.ds(g * CH, CH)],
                        outh_hbm.at[cid, pl.ds(g * CH, CH)])
        pltpu.sync_copy(accd_sh.at[pl.ds(g * CH, CH)],
                        outd_hbm.at[cid, pl.ds(g * CH, CH)])


def kernel(x, edge_index, W_gat, attn_l, attn_r, bias_gat, W_lin):
    ei = edge_index.astype(jnp.int32)
    sd = ei.reshape(2, E // CH, CH).transpose(1, 0, 2)

    z, el, er, m = pl.pallas_call(
        _pre_body,
        out_shape=[
            jax.ShapeDtypeStruct((N, D), jnp.float32),
            jax.ShapeDtypeStruct((N,), jnp.float32),
            jax.ShapeDtypeStruct((N,), jnp.float32),
            jax.ShapeDtypeStruct((LN,), jnp.float32),
        ],
    )(x, W_gat, attn_l, attn_r)

    mesh = plsc.VectorSubcoreMesh(core_axis_name="c", subcore_axis_name="s",
                                  num_cores=NC, num_subcores=NS)
    edge_kernel = pl.kernel(
        _edge_body,
        out_type=[
            jax.ShapeDtypeStruct((NC, N, D), jnp.float32),
            jax.ShapeDtypeStruct((NC, N, LN), jnp.float32),
        ],
        mesh=mesh,
        compiler_params=pltpu.CompilerParams(use_tc_tiling_on_sc=False,
                                             needs_layout_passes=False),
        scratch_types=[
            pltpu.VMEM((N,), jnp.float32),
            pltpu.VMEM((N,), jnp.float32),
            pltpu.VMEM((LN,), jnp.float32),
            pltpu.VMEM((2, CH), jnp.int32),
            pltpu.VMEM((CH, D), jnp.float32),
            pltpu.VMEM((CH, LN), jnp.float32),
            pltpu.VMEM((CH + LN,), jnp.float32),
            pltpu.VMEM_SHARED((N, D), jnp.float32),
            pltpu.VMEM_SHARED((N, LN), jnp.float32),
            pltpu.SemaphoreType.DMA,
            pltpu.SemaphoreType.DMA,
            pltpu.SemaphoreType.DMA,
        ],
    )
    parts_h, parts_d = edge_kernel(z, el, er, sd, m)

    out = pl.pallas_call(
        _post_body,
        out_shape=jax.ShapeDtypeStruct((N, D), jnp.float32),
    )(parts_h, parts_d, bias_gat, W_lin)
    return out


# speedup vs baseline: 29.4690x; 1.2765x over previous
import jax
import jax.numpy as jnp
from jax import lax
from jax.experimental import pallas as pl
from jax.experimental.pallas import tpu as pltpu
from jax.experimental.pallas import tpu_sc as plsc

N = 10000
E = 320000
D = 128
NC = 2
NS = 16
NW = NC * NS
EPW = E // NW
CH = 80
NCHUNK = EPW // CH
LN = 16
HA = 48
HB = CH - HA


def _pre_body(x_ref, wg_ref, al_ref, ar_ref, z_ref, el_ref, er_ref, m_ref):
    z = lax.dot_general(x_ref[...], wg_ref[...], (((1,), (1,)), ((), ())),
                        preferred_element_type=jnp.float32,
                        precision=lax.Precision.HIGHEST)
    z_ref[...] = z
    el = jnp.sum(z * al_ref[...][None, :], axis=1)
    er = jnp.sum(z * ar_ref[...][None, :], axis=1)
    el_ref[...] = el
    er_ref[...] = er
    m = jnp.max(el) + jnp.max(er)
    m = jnp.where(m >= 0.0, m, 0.2 * m)
    m_ref[...] = jnp.full((LN,), m, jnp.float32)


def _post_body(ph_ref, pd_ref, b_ref, wl_ref, o_ref):
    hu = ph_ref[0] + ph_ref[1]
    den = pd_ref[0, :, 0:1] + pd_ref[1, :, 0:1]
    h = hu / (den + 1e-9) + b_ref[...][None, :]
    h = jnp.where(h >= 0.0, h, 0.01 * h)
    o_ref[...] = lax.dot_general(h, wl_ref[...], (((1,), (1,)), ((), ())),
                                 preferred_element_type=jnp.float32,
                                 precision=lax.Precision.HIGHEST)


def _edge_body(z_hbm, el_hbm, er_hbm, sd_hbm, m_hbm,
               outh_hbm, outd_hbm,
               el_v, er_v, m_v, sdA, sdB, zrowA, zrowB, wrowA, wrowB,
               dstA, dstB, w_v,
               acch_sh, accd_sh, gsemA, gsemB, hsemA, hsemB, dsemA, dsemB):
    cid = lax.axis_index("c")
    sid = lax.axis_index("s")
    wid = cid * NS + sid

    pltpu.sync_copy(el_hbm, el_v)
    pltpu.sync_copy(er_hbm, er_v)
    pltpu.sync_copy(m_hbm, m_v)
    mvec = m_v[...]
    lane = lax.iota(jnp.int32, LN)
    zero16 = jnp.zeros((LN,), jnp.float32)

    @pl.loop(0, HA)
    def _zeroA(j):
        for c in range(D // LN):
            zrowA[j, pl.ds(c * LN, LN)] = zero16
        wrowA[j, :] = zero16

    @pl.loop(0, HB)
    def _zeroB(j):
        for c in range(D // LN):
            zrowB[j, pl.ds(c * LN, LN)] = zero16
        wrowB[j, :] = zero16

    @pl.loop(sid, N // CH, step=NS)
    def _zcopy(g):
        pltpu.sync_copy(zrowA, acch_sh.at[pl.ds(g * CH, HA)])
        pltpu.sync_copy(zrowB, acch_sh.at[pl.ds(g * CH + HA, HB)])
        pltpu.sync_copy(wrowA, accd_sh.at[pl.ds(g * CH, HA)])
        pltpu.sync_copy(wrowB, accd_sh.at[pl.ds(g * CH + HA, HB)])

    plsc.subcore_barrier()

    cbase = wid * NCHUNK

    def scale_rows(zrow, nrows, wof, wrow):
        @pl.loop(0, nrows)
        def _scale(j):
            wv = jnp.full((LN,), w_v[pl.ds(wof + j, LN)][0], jnp.float32)
            for c in range(D // LN):
                zrow[j, pl.ds(c * LN, LN)] = zrow[j, pl.ds(c * LN, LN)] * wv
            wrow[j, :] = jnp.where(lane == 0, wv, zero16)

    def chunk_body(ci, sdP, sdQ, prev_pred, has_next):
        if has_next:
            pltpu.sync_copy(sd_hbm.at[cbase + ci + 1], sdQ)

        for g in range(CH // LN):
            si = sdP[0, pl.ds(g * LN, LN)]
            di = sdP[1, pl.ds(g * LN, LN)]
            s = plsc.load_gather(el_v, [si]) + plsc.load_gather(er_v, [di])
            e = jnp.where(s >= 0.0, s, 0.2 * s)
            w_v[pl.ds(g * LN, LN)] = jnp.exp(e - mvec)

        def drain_prev_b():
            pltpu.make_async_copy(zrowB, acch_sh.at[dstB], hsemB).wait()
            pltpu.make_async_copy(wrowB, accd_sh.at[dstB], dsemB).wait()

        if prev_pred is None:
            drain_prev_b()
        else:
            pl.when(prev_pred)(drain_prev_b)

        for g in range(HA // LN):
            dstA[pl.ds(g * LN, LN)] = sdP[1, pl.ds(g * LN, LN)]
        for g in range(HB // LN):
            dstB[pl.ds(g * LN, LN)] = sdP[1, pl.ds(HA + g * LN, LN)]

        pltpu.make_async_copy(z_hbm.at[sdP.at[0, pl.ds(0, HA)]],
                              zrowA, gsemA).wait()
        pltpu.async_copy(z_hbm.at[sdP.at[0, pl.ds(HA, HB)]], zrowB, gsemB)

        scale_rows(zrowA, HA, 0, wrowA)
        pltpu.async_copy(zrowA, acch_sh.at[dstA], hsemA, add=True)
        pltpu.async_copy(wrowA, accd_sh.at[dstA], dsemA, add=True)

        pltpu.make_async_copy(z_hbm.at[sdP.at[0, pl.ds(HA, HB)]],
                              zrowB, gsemB).wait()
        scale_rows(zrowB, HB, HA, wrowB)

        pltpu.make_async_copy(zrowA, acch_sh.at[dstA], hsemA).wait()
        pltpu.make_async_copy(wrowA, accd_sh.at[dstA], dsemA).wait()
        if has_next:
            pltpu.async_copy(z_hbm.at[sdQ.at[0, pl.ds(0, HA)]], zrowA, gsemA)

        pltpu.async_copy(zrowB, acch_sh.at[dstB], hsemB, add=True)
        pltpu.async_copy(wrowB, accd_sh.at[dstB], dsemB, add=True)

    pltpu.sync_copy(sd_hbm.at[cbase], sdA)
    pltpu.async_copy(z_hbm.at[sdA.at[0, pl.ds(0, HA)]], zrowA, gsemA)

    @pl.loop(0, NCHUNK // 2)
    def _pair(k):
        chunk_body(2 * k, sdA, sdB, k > 0, True)
        chunk_body(2 * k + 1, sdB, sdA, None, True)

    chunk_body(NCHUNK - 1, sdA, sdB, None, False)
    pltpu.make_async_copy(zrowB, acch_sh.at[dstB], hsemB).wait()
    pltpu.make_async_copy(wrowB, accd_sh.at[dstB], dsemB).wait()

    plsc.subcore_barrier()

    @pl.loop(sid, N // CH, step=NS)
    def _out(g):
        pltpu.sync_copy(acch_sh.at[pl.ds(g * CH, CH)],
                        outh_hbm.at[cid, pl.ds(g * CH, CH)])
        pltpu.sync_copy(accd_sh.at[pl.ds(g * CH, CH)],
                        outd_hbm.at[cid, pl.ds(g * CH, CH)])


def kernel(x, edge_index, W_gat, attn_l, attn_r, bias_gat, W_lin):
    ei = edge_index.astype(jnp.int32)
    sd = ei.reshape(2, E // CH, CH).transpose(1, 0, 2)

    z, el, er, m = pl.pallas_call(
        _pre_body,
        out_shape=[
            jax.ShapeDtypeStruct((N, D), jnp.float32),
            jax.ShapeDtypeStruct((N,), jnp.float32),
            jax.ShapeDtypeStruct((N,), jnp.float32),
            jax.ShapeDtypeStruct((LN,), jnp.float32),
        ],
    )(x, W_gat, attn_l, attn_r)

    mesh = plsc.VectorSubcoreMesh(core_axis_name="c", subcore_axis_name="s",
                                  num_cores=NC, num_subcores=NS)
    edge_kernel = pl.kernel(
        _edge_body,
        out_type=[
            jax.ShapeDtypeStruct((NC, N, D), jnp.float32),
            jax.ShapeDtypeStruct((NC, N, LN), jnp.float32),
        ],
        mesh=mesh,
        compiler_params=pltpu.CompilerParams(use_tc_tiling_on_sc=False,
                                             needs_layout_passes=False),
        scratch_types=[
            pltpu.VMEM((N,), jnp.float32),
            pltpu.VMEM((N,), jnp.float32),
            pltpu.VMEM((LN,), jnp.float32),
            pltpu.VMEM((2, CH), jnp.int32),
            pltpu.VMEM((2, CH), jnp.int32),
            pltpu.VMEM((HA, D), jnp.float32),
            pltpu.VMEM((HB, D), jnp.float32),
            pltpu.VMEM((HA, LN), jnp.float32),
            pltpu.VMEM((HB, LN), jnp.float32),
            pltpu.VMEM((HA,), jnp.int32),
            pltpu.VMEM((HB,), jnp.int32),
            pltpu.VMEM((CH + LN,), jnp.float32),
            pltpu.VMEM_SHARED((N, D), jnp.float32),
            pltpu.VMEM_SHARED((N, LN), jnp.float32),
            pltpu.SemaphoreType.DMA,
            pltpu.SemaphoreType.DMA,
            pltpu.SemaphoreType.DMA,
            pltpu.SemaphoreType.DMA,
            pltpu.SemaphoreType.DMA,
            pltpu.SemaphoreType.DMA,
        ],
    )
    parts_h, parts_d = edge_kernel(z, el, er, sd, m)

    out = pl.pallas_call(
        _post_body,
        out_shape=jax.ShapeDtypeStruct((N, D), jnp.float32),
    )(parts_h, parts_d, bias_gat, W_lin)
    return out
